# Initial kernel scaffold; baseline (speedup 1.0000x reference)
import jax
import jax.numpy as jnp
from jax.experimental import pallas as pl


def kernel(x, edge_index, edge_weight, Wz, bz, Wr, br, Wh, bh, W_lin, b_lin):
    raise NotImplementedError("write your pallas kernel here")


# speedup vs baseline: 1.3965x; 1.3965x over previous
import jax
import jax.numpy as jnp
from jax.experimental import pallas as pl

_BLOCK_ROWS = 1000


def _fused_gru_readout(x_ref, wz_ref, wh_ref, bz_ref, bh_ref, wl_ref, bl_ref,
                       o_ref):
    xb = x_ref[...]
    pre_z = jnp.dot(xb, wz_ref[...], preferred_element_type=jnp.float32)
    pre_h = jnp.dot(xb, wh_ref[...], preferred_element_type=jnp.float32)
    z = jax.nn.sigmoid(pre_z + bz_ref[...])
    ht = jnp.tanh(pre_h + bh_ref[...])
    h = jnp.maximum((1.0 - z) * ht, 0.0)
    o_ref[...] = jnp.sum(h * wl_ref[...], axis=1, keepdims=True) + bl_ref[...]


def kernel(x, edge_index, edge_weight, Wz, bz, Wr, br, Wh, bh, W_lin, b_lin):
    del edge_index, edge_weight, Wr, br
    n, f_in = x.shape
    f_out = W_lin.shape[0]
    wz = (Wz[0, 0, :f_in, :] + Wz[1, 0, :f_in, :]).astype(jnp.float32)
    wh = (Wh[0, 0, :f_in, :] + Wh[1, 0, :f_in, :]).astype(jnp.float32)
    bz2 = bz.reshape(1, f_out)
    bh2 = bh.reshape(1, f_out)
    wl2 = W_lin.reshape(1, f_out)
    bl2 = b_lin.reshape(1, 1)

    grid = (n // _BLOCK_ROWS,)
    fixed = lambda i: (0, 0)
    out = pl.pallas_call(
        _fused_gru_readout,
        grid=grid,
        in_specs=[
            pl.BlockSpec((_BLOCK_ROWS, f_in), lambda i: (i, 0)),
            pl.BlockSpec((f_in, f_out), fixed),
            pl.BlockSpec((f_in, f_out), fixed),
            pl.BlockSpec((1, f_out), fixed),
            pl.BlockSpec((1, f_out), fixed),
            pl.BlockSpec((1, f_out), fixed),
            pl.BlockSpec((1, 1), fixed),
        ],
        out_specs=pl.BlockSpec((_BLOCK_ROWS, 1), lambda i: (i, 0)),
        out_shape=jax.ShapeDtypeStruct((n, 1), jnp.float32),
    )(x, wz, wh, bz2, bh2, wl2, bl2)
    return out


# speedup vs baseline: 1.6412x; 1.1752x over previous
import jax
import jax.numpy as jnp
from jax.experimental import pallas as pl

_BLOCK_ROWS = 2000


def _fused_gru_readout(x_ref, wz_ref, wh_ref, bz_ref, bh_ref, wl_ref, bl_ref,
                       o_ref):
    xb = x_ref[...]
    pre_z = jnp.dot(xb, wz_ref[...], preferred_element_type=jnp.float32)
    pre_h = jnp.dot(xb, wh_ref[...], preferred_element_type=jnp.float32)
    z = jax.nn.sigmoid(pre_z + bz_ref[...])
    ht = jnp.tanh(pre_h + bh_ref[...])
    h = jnp.maximum((1.0 - z) * ht, 0.0)
    o_ref[...] = jnp.sum(h * wl_ref[...], axis=1, keepdims=True) + bl_ref[...]


def kernel(x, edge_index, edge_weight, Wz, bz, Wr, br, Wh, bh, W_lin, b_lin):
    del edge_index, edge_weight, Wr, br
    n, f_in = x.shape
    f_out = W_lin.shape[0]
    wz = (Wz[0, 0, :f_in, :] + Wz[1, 0, :f_in, :]).astype(jnp.float32)
    wh = (Wh[0, 0, :f_in, :] + Wh[1, 0, :f_in, :]).astype(jnp.float32)
    bz2 = bz.reshape(1, f_out)
    bh2 = bh.reshape(1, f_out)
    wl2 = W_lin.reshape(1, f_out)
    bl2 = b_lin.reshape(1, 1)

    grid = (n // _BLOCK_ROWS,)
    fixed = lambda i: (0, 0)
    out = pl.pallas_call(
        _fused_gru_readout,
        grid=grid,
        in_specs=[
            pl.BlockSpec((_BLOCK_ROWS, f_in), lambda i: (i, 0)),
            pl.BlockSpec((f_in, f_out), fixed),
            pl.BlockSpec((f_in, f_out), fixed),
            pl.BlockSpec((1, f_out), fixed),
            pl.BlockSpec((1, f_out), fixed),
            pl.BlockSpec((1, f_out), fixed),
            pl.BlockSpec((1, 1), fixed),
        ],
        out_specs=pl.BlockSpec((_BLOCK_ROWS, 1), lambda i: (i, 0)),
        out_shape=jax.ShapeDtypeStruct((n, 1), jnp.float32),
    )(x, wz, wh, bz2, bh2, wl2, bl2)
    return out


# speedup vs baseline: 1.7950x; 1.0937x over previous
import jax
import jax.numpy as jnp
from jax.experimental import pallas as pl

_BLOCK_ROWS = 5000


def _fused_gru_readout(x_ref, wz_ref, wh_ref, bz_ref, bh_ref, wl_ref, bl_ref,
                       o_ref):
    xb = x_ref[...]
    pre_z = jnp.dot(xb, wz_ref[...], preferred_element_type=jnp.float32)
    pre_h = jnp.dot(xb, wh_ref[...], preferred_element_type=jnp.float32)
    z = jax.nn.sigmoid(pre_z + bz_ref[...])
    ht = jnp.tanh(pre_h + bh_ref[...])
    h = jnp.maximum((1.0 - z) * ht, 0.0)
    o_ref[...] = jnp.sum(h * wl_ref[...], axis=1, keepdims=True) + bl_ref[...]


def kernel(x, edge_index, edge_weight, Wz, bz, Wr, br, Wh, bh, W_lin, b_lin):
    del edge_index, edge_weight, Wr, br
    n, f_in = x.shape
    f_out = W_lin.shape[0]
    wz = (Wz[0, 0, :f_in, :] + Wz[1, 0, :f_in, :]).astype(jnp.float32)
    wh = (Wh[0, 0, :f_in, :] + Wh[1, 0, :f_in, :]).astype(jnp.float32)
    bz2 = bz.reshape(1, f_out)
    bh2 = bh.reshape(1, f_out)
    wl2 = W_lin.reshape(1, f_out)
    bl2 = b_lin.reshape(1, 1)

    grid = (n // _BLOCK_ROWS,)
    fixed = lambda i: (0, 0)
    out = pl.pallas_call(
        _fused_gru_readout,
        grid=grid,
        in_specs=[
            pl.BlockSpec((_BLOCK_ROWS, f_in), lambda i: (i, 0)),
            pl.BlockSpec((f_in, f_out), fixed),
            pl.BlockSpec((f_in, f_out), fixed),
            pl.BlockSpec((1, f_out), fixed),
            pl.BlockSpec((1, f_out), fixed),
            pl.BlockSpec((1, f_out), fixed),
            pl.BlockSpec((1, 1), fixed),
        ],
        out_specs=pl.BlockSpec((_BLOCK_ROWS, 1), lambda i: (i, 0)),
        out_shape=jax.ShapeDtypeStruct((n, 1), jnp.float32),
    )(x, wz, wh, bz2, bh2, wl2, bl2)
    return out
